# baseline (speedup 1.0000x reference)
import jax
import jax.numpy as jnp
from jax import lax
from jax.experimental import pallas as pl
from jax.experimental.pallas import tpu as pltpu
from jax.experimental.pallas import tpu_sc as plsc

_N = 10000
_E = 320000
_D = 128
_NC = 2
_NS = 16
_NW = _NC * _NS
_CHUNK = 80
_NCHUNK = 125
_NP = 10240
_RPT = _NP // _NS


def _mlp_body(x_ref, w1, b1, w2, b2, w3, b3, w4, b4, x1_ref, m2_ref):
    xb = x_ref[...]
    dn = (((1,), (1,)), ((), ()))
    h = jnp.maximum(
        lax.dot_general(xb, w1[...], dn, preferred_element_type=jnp.float32)
        + b1[...], 0.0)
    x1_ref[...] = jnp.maximum(
        lax.dot_general(h, w2[...], dn, preferred_element_type=jnp.float32)
        + b2[...], 0.0)
    g = jnp.maximum(
        lax.dot_general(xb, w3[...], dn, preferred_element_type=jnp.float32)
        + b3[...], 0.0)
    m2_ref[...] = jnp.maximum(
        lax.dot_general(g, w4[...], dn, preferred_element_type=jnp.float32)
        + b4[...], 0.0)


def _mlps(x2d, W1, b1, W2, b2, W3, b3, W4, b4):
    BM = 2000
    wspec = pl.BlockSpec((_D, _D), lambda i: (0, 0))
    bspec = pl.BlockSpec((1, _D), lambda i: (0, 0))
    rspec = pl.BlockSpec((BM, _D), lambda i: (i, 0))
    return pl.pallas_call(
        _mlp_body,
        grid=(_N // BM,),
        in_specs=[rspec, wspec, bspec, wspec, bspec, wspec, bspec, wspec, bspec],
        out_specs=[rspec, rspec],
        out_shape=[jax.ShapeDtypeStruct((_N, _D), jnp.float32)] * 2,
    )(x2d, W1, b1.reshape(1, _D), W2, b2.reshape(1, _D),
      W3, b3.reshape(1, _D), W4, b4.reshape(1, _D))


def _add2_body(a_ref, b_ref, o_ref):
    o_ref[...] = a_ref[...] + b_ref[...]


def _add3_body(a_ref, b_ref, c_ref, o_ref):
    o_ref[...] = a_ref[...] + b_ref[...] + c_ref[...]


def _combine(parts, extra=None):
    BM = 2000
    rspec = pl.BlockSpec((BM, _D), lambda i: (i, 0))
    args = [parts[0], parts[1]] + ([] if extra is None else [extra])
    body = _add2_body if extra is None else _add3_body
    return pl.pallas_call(
        body,
        grid=(_N // BM,),
        in_specs=[rspec] * len(args),
        out_specs=rspec,
        out_shape=jax.ShapeDtypeStruct((_N, _D), jnp.float32),
    )(*args)


def _spmm_body(meta_hbm, x_hbm, out_hbm,
               meta, srci, dsti, rows, acc_sh, gsem, ssem, msem):
    c = lax.axis_index("c")
    s = lax.axis_index("s")
    w = c * _NS + s

    zvec = jnp.zeros((16,), jnp.float32)

    def zbody(i, carry):
        rows[0][i // 8, pl.ds((i % 8) * 16, 16)] = zvec
        return carry

    lax.fori_loop(0, _CHUNK * 8, zbody, 0)
    r0 = s * _RPT
    for t in range(_RPT // _CHUNK):
        pltpu.sync_copy(rows[0], acc_sh.at[pl.ds(r0 + t * _CHUNK, _CHUNK)])
    plsc.subcore_barrier()

    def meta_load(j, b):
        pltpu.async_copy(meta_hbm.at[w].at[j], meta[b], msem[b])

    def wait_meta(j, b):
        pltpu.make_async_copy(meta_hbm.at[w].at[j], meta[b], msem[b]).wait()

    def cvt_idx(b):
        for g in range(_CHUNK // 16):
            sl = pl.ds(g * 16, 16)
            srci[b][0, sl] = meta[b][0, sl].astype(jnp.int32)
            dsti[b][0, sl] = meta[b][1, sl].astype(jnp.int32)

    def gather(j, b):
        pltpu.async_copy(x_hbm.at[srci[b].at[0]], rows[b], gsem[b])

    def wait_gather(j, b):
        pltpu.make_async_copy(
            x_hbm.at[srci[b].at[0]], rows[b], gsem[b]).wait()

    def scatter(b):
        pltpu.async_copy(rows[b], acc_sh.at[dsti[b].at[0]], ssem[b], add=True)

    def wait_scatter(b):
        pltpu.make_async_copy(
            rows[b], acc_sh.at[dsti[b].at[0]], ssem[b]).wait()

    def scale(b):
        r = rows[b]
        for g in range(_CHUNK // 16):
            vv = meta[b][2, pl.ds(g * 16, 16)]
            for l in range(16):
                e = g * 16 + l
                v = vv[l]
                for k in range(_D // 16):
                    r[e, pl.ds(k * 16, 16)] = r[e, pl.ds(k * 16, 16)] * v

    meta_load(0, 0)
    wait_meta(0, 0)
    cvt_idx(0)
    gather(0, 0)
    meta_load(1, 1)
    nq = (_NCHUNK - 2) // 3

    def chunk_step(j, b, launch=True, load_next=True):
        nb = (b + 1) % 3

        if isinstance(j, int):
            if j >= 2:
                wait_scatter(nb)
        else:
            @pl.when(j >= 2)
            def _():
                wait_scatter(nb)

        if launch:
            wait_meta(j + 1, nb)
            cvt_idx(nb)
            gather(j + 1, nb)
            if load_next:
                meta_load(j + 2, (nb + 1) % 3)

        wait_gather(j, b)
        scale(b)
        scatter(b)

    def body(t, carry):
        for q in range(3):
            chunk_step(3 * t + q, q)
        return carry

    lax.fori_loop(0, nq, body, 0)
    chunk_step(_NCHUNK - 2, 0, load_next=False)
    chunk_step(_NCHUNK - 1, 1, launch=False)
    wait_scatter(0)
    wait_scatter(1)
    plsc.subcore_barrier()
    pltpu.sync_copy(acc_sh.at[pl.ds(r0, _RPT)],
                    out_hbm.at[pl.ds(c * _NP + r0, _RPT)])


def _spmm_partials(meta, x_mat):
    mesh = plsc.VectorSubcoreMesh(
        core_axis_name="c", subcore_axis_name="s",
        num_cores=_NC, num_subcores=_NS)
    kern = pl.kernel(
        _spmm_body,
        out_type=jax.ShapeDtypeStruct((_NC * _NP, _D), jnp.float32),
        mesh=mesh,
        scratch_types=[
            [pltpu.VMEM((3, _CHUNK), jnp.float32)] * 3,
            [pltpu.VMEM((1, _CHUNK), jnp.int32)] * 3,
            [pltpu.VMEM((1, _CHUNK), jnp.int32)] * 3,
            [pltpu.VMEM((_CHUNK, _D), jnp.float32)] * 3,
            pltpu.VMEM_SHARED((_NP, _D), jnp.float32),
            [pltpu.SemaphoreType.DMA] * 3,
            [pltpu.SemaphoreType.DMA] * 3,
            [pltpu.SemaphoreType.DMA] * 3,
        ],
    )
    return kern(meta, x_mat)


def _edge_meta(idx, val):
    srcf = idx[1].astype(jnp.float32).reshape(_NW, _NCHUNK, 1, _CHUNK)
    dstf = idx[0].astype(jnp.float32).reshape(_NW, _NCHUNK, 1, _CHUNK)
    vf = val.reshape(_NW, _NCHUNK, 1, _CHUNK)
    return jnp.concatenate([srcf, dstf, vf], axis=2)


def kernel(K_value, index, normed_A_value, A_index, x, n1, n2,
           W1, b1, W2, b2, W3, b3, W4, b4):
    x2d = x.reshape(_N, _D)
    x1, m2 = _mlps(x2d, W1, b1, W2, b2, W3, b3, W4, b4)

    p = _spmm_partials(_edge_meta(index, K_value), x1)
    wx = _combine((p[:_N], p[_NP:_NP + _N]))

    q = _spmm_partials(_edge_meta(A_index, normed_A_value), wx)
    out = _combine((q[:_N], q[_NP:_NP + _N]), extra=m2)
    return out[None]
